# baseline (speedup 1.0000x reference)
import functools

import jax
import jax.numpy as jnp
from jax import lax
from jax.experimental import pallas as pl
from jax.experimental.pallas import tpu as pltpu
from jax.experimental.pallas import tpu_sc as plsc

EMBED = 1024
NE = 8
CAP = 1024
DFF = 4096
T = 8192

_HI = jax.lax.Precision.DEFAULT


_RBM = 256


def _router_body(x_ref, wr1_ref, br1_ref, wr2_ref, br2_ref,
                 sel_ref, gate_ref, cnt_ref, rankl_ref):
    h = jnp.maximum(
        jnp.dot(x_ref[...], wr1_ref[...], precision=_HI) + br1_ref[...], 0.0)
    logits = jnp.dot(h, wr2_ref[...], precision=_HI) + br2_ref[...]
    lmax = jnp.max(logits, axis=1, keepdims=True)
    iota8 = jax.lax.broadcasted_iota(jnp.int32, logits.shape, 1)
    sel = jnp.min(jnp.where(logits >= lmax, iota8, NE), axis=1)
    gate = 1.0 / jnp.sum(jnp.exp(logits - lmax), axis=1)
    sel_ref[...] = sel[:, None]
    gate_ref[...] = gate[:, None]
    onehot = (sel[:, None] == jax.lax.broadcasted_iota(
        jnp.int32, (_RBM, 16), 1)).astype(jnp.int32)
    cnt_ref[...] = jnp.sum(onehot, axis=0, keepdims=True)[None]
    cum = onehot
    sh = 1
    while sh < _RBM:
        cum = cum + jnp.concatenate(
            [jnp.zeros((sh, 16), jnp.int32), cum[:-sh]], axis=0)
        sh *= 2
    rankl_ref[...] = jnp.sum(onehot * (cum - 1), axis=1)[:, None]


def _router(x, Wr1, br1, Wr2, br2):
    grid = (T // _RBM,)
    return pl.pallas_call(
        _router_body,
        grid=grid,
        in_specs=[
            pl.BlockSpec((_RBM, EMBED), lambda i: (i, 0)),
            pl.BlockSpec((EMBED, EMBED), lambda i: (0, 0)),
            pl.BlockSpec((1, EMBED), lambda i: (0, 0)),
            pl.BlockSpec((EMBED, NE), lambda i: (0, 0)),
            pl.BlockSpec((1, NE), lambda i: (0, 0)),
        ],
        out_specs=[
            pl.BlockSpec((_RBM, 1), lambda i: (i, 0)),
            pl.BlockSpec((_RBM, 1), lambda i: (i, 0)),
            pl.BlockSpec((1, 1, 16), lambda i: (i, 0, 0)),
            pl.BlockSpec((_RBM, 1), lambda i: (i, 0)),
        ],
        out_shape=[
            jax.ShapeDtypeStruct((T, 1), jnp.int32),
            jax.ShapeDtypeStruct((T, 1), jnp.float32),
            jax.ShapeDtypeStruct((T // _RBM, 1, 16), jnp.int32),
            jax.ShapeDtypeStruct((T, 1), jnp.int32),
        ],
    )(x, Wr1, br1.reshape(1, EMBED), Wr2, br2.reshape(1, NE))


_BF = 1024
_NF = DFF // _BF


def _ffn_body(xs_ref, w1_ref, b1_ref, w2_ref, b2_ref, g_ref, out_ref):
    f = pl.program_id(1)
    h = jnp.maximum(
        jnp.dot(xs_ref[...], w1_ref[0], precision=_HI) + b1_ref[0], 0.0)
    contrib = jnp.dot(h, w2_ref[0], precision=_HI)

    @pl.when(f == 0)
    def _():
        out_ref[...] = contrib + b2_ref[0]

    @pl.when(f != 0)
    def _():
        out_ref[...] = out_ref[...] + contrib

    @pl.when(f == _NF - 1)
    def _():
        out_ref[...] = out_ref[...] * g_ref[...]


def _ffn(x_sorted, W1, b1, W2, b2, gate_sorted):
    grid = (NE, _NF)
    return pl.pallas_call(
        _ffn_body,
        grid=grid,
        in_specs=[
            pl.BlockSpec((CAP, EMBED), lambda e, f: (e, 0)),
            pl.BlockSpec((1, EMBED, _BF), lambda e, f: (e, 0, f)),
            pl.BlockSpec((1, 1, _BF), lambda e, f: (e, 0, f)),
            pl.BlockSpec((1, _BF, EMBED), lambda e, f: (e, f, 0)),
            pl.BlockSpec((1, 1, EMBED), lambda e, f: (e, 0, 0)),
            pl.BlockSpec((CAP, 1), lambda e, f: (e, 0)),
        ],
        out_specs=pl.BlockSpec((CAP, EMBED), lambda e, f: (e, 0)),
        out_shape=jax.ShapeDtypeStruct((T, EMBED), jnp.float32),
        compiler_params=pltpu.CompilerParams(
            dimension_semantics=("parallel", "arbitrary")),
    )(x_sorted, W1, b1.reshape(NE, 1, DFF), W2, b2.reshape(NE, 1, EMBED),
      gate_sorted)


_NC = 2
_NS = 16
_NW = _NC * _NS
_L = 16
_CHUNK = T // _NW
_NV = _CHUNK // _L

_MESH = dict(core_axis_name="c", subcore_axis_name="s")


def _take16(vec, idx):
    return vec.at[idx].get(mode="promise_in_bounds")


def _cumsum16(x):
    lane = lax.iota(jnp.int32, _L)
    cs = x
    for sh in (1, 2, 4, 8):
        shifted = _take16(cs, jnp.maximum(lane - sh, 0))
        cs = cs + jnp.where(lane >= sh, shifted, jnp.zeros_like(cs))
    return cs


def _splat16(vec, e):
    return _take16(vec, jnp.full((_L,), e, jnp.int32))


def _routing_body(sel_hbm, gate_hbm, cnt_hbm, rankl_hbm,
                  disp_hbm, gsort_hbm, comb_hbm,
                  cnt_v, sel_v, gate_v, rankl_v, comb_v, tok2d, g2d, d2d,
                  scat_sems):
    wid = lax.axis_index("s") * _NC + lax.axis_index("c")
    base_tok = wid * _CHUNK
    lane = lax.iota(jnp.int32, _L)
    zero = jnp.zeros((_L,), jnp.int32)

    pltpu.sync_copy(cnt_hbm, cnt_v)
    pltpu.sync_copy(sel_hbm.at[pl.ds(base_tok, _CHUNK)], sel_v)
    pltpu.sync_copy(gate_hbm.at[pl.ds(base_tok, _CHUNK)], gate_v)
    pltpu.sync_copy(rankl_hbm.at[pl.ds(base_tok, _CHUNK)], rankl_v)

    base = zero
    tot = zero
    for w in range(_NW):
        row = cnt_v[w]
        base = base + jnp.where(w < wid, row, zero)
        tot = tot + row

    expert_lane = lane < NE
    drops = jnp.maximum(tot - CAP, 0)
    pads = jnp.where(expert_lane, jnp.maximum(CAP - tot, 0), 0)
    drop_excl = _cumsum16(drops) - drops
    pad_incl = _cumsum16(pads)
    pad_excl = pad_incl - pads

    for v in range(_NV):
        s = sel_v[pl.ds(v * _L, _L)]
        g = gate_v[pl.ds(v * _L, _L)]
        tok = base_tok + v * _L + lane
        rank = _take16(base, s) + rankl_v[pl.ds(v * _L, _L)]

        kept = rank < CAP
        kslot = s * CAP + rank
        kd = _take16(drop_excl, s) + (rank - CAP)
        ep = zero
        for e in range(NE):
            ep = ep + jnp.where(kd >= _splat16(pad_incl, e),
                                jnp.ones_like(kd), jnp.zeros_like(kd))
        ep = jnp.minimum(ep, NE - 1)
        slot_pad = ep * CAP + _take16(tot, ep) + (kd - _take16(pad_excl, ep))
        d = jnp.where(kept, kslot, slot_pad)

        j, c = v // (_NV // 2), (v % (_NV // 2)) * _L
        comb_v[pl.ds(v * _L, _L)] = d
        d2d[j, pl.ds(c, _L)] = d
        tok2d[j, pl.ds(c, _L)] = tok
        g2d[j, pl.ds(c, _L)] = jnp.where(kept, g, 0.0)

    copies = []
    for j in range(2):
        copies.append(pltpu.make_async_copy(
            tok2d.at[j], disp_hbm.at[d2d.at[j]], scat_sems.at[0]))
        copies.append(pltpu.make_async_copy(
            g2d.at[j], gsort_hbm.at[d2d.at[j]], scat_sems.at[1]))
    for c in copies:
        c.start()
    for c in copies:
        c.wait()
    pltpu.sync_copy(comb_v, comb_hbm.at[pl.ds(base_tok, _CHUNK)])


def _sc_route(sel, gate, counts, rankl):
    f = functools.partial(
        pl.kernel,
        out_type=[
            jax.ShapeDtypeStruct((T,), jnp.int32),
            jax.ShapeDtypeStruct((T,), jnp.float32),
            jax.ShapeDtypeStruct((T,), jnp.int32),
        ],
        mesh=plsc.VectorSubcoreMesh(**_MESH),
        scratch_types=[
            pltpu.VMEM((_NW, 16), jnp.int32),
            pltpu.VMEM((_CHUNK,), jnp.int32),
            pltpu.VMEM((_CHUNK,), jnp.float32),
            pltpu.VMEM((_CHUNK,), jnp.int32),
            pltpu.VMEM((_CHUNK,), jnp.int32),
            pltpu.VMEM((2, 128), jnp.int32),
            pltpu.VMEM((2, 128), jnp.float32),
            pltpu.VMEM((2, 128), jnp.int32),
            pltpu.SemaphoreType.DMA((2,)),
        ],
    )(_routing_body)
    return f(sel, gate, counts, rankl)


_GC = 64


def _gather_body(src_hbm, idx_hbm, out_hbm, idx_v, rows_v, sem):
    wid = lax.axis_index("s") * _NC + lax.axis_index("c")
    base = wid * _CHUNK
    pltpu.sync_copy(idx_hbm.at[pl.ds(base, _CHUNK)], idx_v)
    for c in range(_CHUNK // _GC):
        pltpu.async_copy(
            src_hbm.at[idx_v.at[pl.ds(c * _GC, _GC)]], rows_v, sem).wait()
        pltpu.sync_copy(rows_v, out_hbm.at[pl.ds(base + c * _GC, _GC)])


def _sc_gather_rows(src, idx):
    f = functools.partial(
        pl.kernel,
        out_type=jax.ShapeDtypeStruct((T, EMBED), jnp.float32),
        mesh=plsc.VectorSubcoreMesh(**_MESH),
        scratch_types=[
            pltpu.VMEM((_CHUNK,), jnp.int32),
            pltpu.VMEM((_GC, EMBED), jnp.float32),
            pltpu.SemaphoreType.DMA,
        ],
    )(_gather_body)
    return f(src, idx)


def kernel(x, Wr1, br1, Wr2, br2, W1, b1, W2, b2):
    sel2, gate2, counts16, rankl2 = _router(x, Wr1, br1, Wr2, br2)
    disp, gsort, comb = _sc_route(sel2[:, 0], gate2[:, 0],
                                  counts16.reshape(_NW, 16), rankl2[:, 0])
    x_sorted = _sc_gather_rows(x, disp)
    y = _ffn(x_sorted, W1, b1, W2, b2, gsort[:, None])
    return _sc_gather_rows(y, comb)


# speedup vs baseline: 1.1229x; 1.1229x over previous
import functools

import jax
import jax.numpy as jnp
from jax import lax
from jax.experimental import pallas as pl
from jax.experimental.pallas import tpu as pltpu
from jax.experimental.pallas import tpu_sc as plsc

EMBED = 1024
NE = 8
CAP = 1024
DFF = 4096
T = 8192

_HI = jax.lax.Precision.DEFAULT


_RBM = 256


def _router_body(x_ref, wr1_ref, br1_ref, wr2_ref, br2_ref,
                 sel_ref, gate_ref, cnt_ref, rankl_ref):
    h = jnp.maximum(
        jnp.dot(x_ref[...], wr1_ref[...], precision=_HI) + br1_ref[...], 0.0)
    logits = jnp.dot(h, wr2_ref[...], precision=_HI) + br2_ref[...]
    lmax = jnp.max(logits, axis=1, keepdims=True)
    iota8 = jax.lax.broadcasted_iota(jnp.int32, logits.shape, 1)
    sel = jnp.min(jnp.where(logits >= lmax, iota8, NE), axis=1)
    gate = 1.0 / jnp.sum(jnp.exp(logits - lmax), axis=1)
    sel_ref[...] = sel[:, None]
    gate_ref[...] = gate[:, None]
    onehot = (sel[:, None] == jax.lax.broadcasted_iota(
        jnp.int32, (_RBM, 16), 1)).astype(jnp.int32)
    cnt_ref[...] = jnp.sum(onehot, axis=0, keepdims=True)[None]
    cum = onehot
    sh = 1
    while sh < _RBM:
        cum = cum + jnp.concatenate(
            [jnp.zeros((sh, 16), jnp.int32), cum[:-sh]], axis=0)
        sh *= 2
    rankl_ref[...] = jnp.sum(onehot * (cum - 1), axis=1)[:, None]


def _router(x, Wr1, br1, Wr2, br2):
    grid = (T // _RBM,)
    return pl.pallas_call(
        _router_body,
        grid=grid,
        in_specs=[
            pl.BlockSpec((_RBM, EMBED), lambda i: (i, 0)),
            pl.BlockSpec((EMBED, EMBED), lambda i: (0, 0)),
            pl.BlockSpec((1, EMBED), lambda i: (0, 0)),
            pl.BlockSpec((EMBED, NE), lambda i: (0, 0)),
            pl.BlockSpec((1, NE), lambda i: (0, 0)),
        ],
        out_specs=[
            pl.BlockSpec((_RBM, 1), lambda i: (i, 0)),
            pl.BlockSpec((_RBM, 1), lambda i: (i, 0)),
            pl.BlockSpec((1, 1, 16), lambda i: (i, 0, 0)),
            pl.BlockSpec((_RBM, 1), lambda i: (i, 0)),
        ],
        out_shape=[
            jax.ShapeDtypeStruct((T, 1), jnp.int32),
            jax.ShapeDtypeStruct((T, 1), jnp.float32),
            jax.ShapeDtypeStruct((T // _RBM, 1, 16), jnp.int32),
            jax.ShapeDtypeStruct((T, 1), jnp.int32),
        ],
    )(x, Wr1, br1.reshape(1, EMBED), Wr2, br2.reshape(1, NE))


_BF = 1024
_NF = DFF // _BF


def _ffn_body(xs_ref, w1_ref, b1_ref, w2_ref, b2_ref, g_ref, out_ref):
    f = pl.program_id(1)
    h = jnp.maximum(
        jnp.dot(xs_ref[...], w1_ref[0], precision=_HI) + b1_ref[0], 0.0)
    contrib = jnp.dot(h, w2_ref[0], precision=_HI)

    @pl.when(f == 0)
    def _():
        out_ref[...] = contrib + b2_ref[0]

    @pl.when(f != 0)
    def _():
        out_ref[...] = out_ref[...] + contrib

    @pl.when(f == _NF - 1)
    def _():
        out_ref[...] = out_ref[...] * g_ref[...]


def _ffn(x_sorted, W1, b1, W2, b2, gate_sorted):
    grid = (NE, _NF)
    return pl.pallas_call(
        _ffn_body,
        grid=grid,
        in_specs=[
            pl.BlockSpec((CAP, EMBED), lambda e, f: (e, 0)),
            pl.BlockSpec((1, EMBED, _BF), lambda e, f: (e, 0, f)),
            pl.BlockSpec((1, 1, _BF), lambda e, f: (e, 0, f)),
            pl.BlockSpec((1, _BF, EMBED), lambda e, f: (e, f, 0)),
            pl.BlockSpec((1, 1, EMBED), lambda e, f: (e, 0, 0)),
            pl.BlockSpec((CAP, 1), lambda e, f: (e, 0)),
        ],
        out_specs=pl.BlockSpec((CAP, EMBED), lambda e, f: (e, 0)),
        out_shape=jax.ShapeDtypeStruct((T, EMBED), jnp.float32),
        compiler_params=pltpu.CompilerParams(
            dimension_semantics=("parallel", "arbitrary")),
    )(x_sorted, W1, b1.reshape(NE, 1, DFF), W2, b2.reshape(NE, 1, EMBED),
      gate_sorted)


_NC = 2
_NS = 16
_NW = _NC * _NS
_L = 16
_CHUNK = T // _NW
_NV = _CHUNK // _L

_MESH = dict(core_axis_name="c", subcore_axis_name="s")


def _take16(vec, idx):
    return vec.at[idx].get(mode="promise_in_bounds")


def _cumsum16(x):
    lane = lax.iota(jnp.int32, _L)
    cs = x
    for sh in (1, 2, 4, 8):
        shifted = _take16(cs, jnp.maximum(lane - sh, 0))
        cs = cs + jnp.where(lane >= sh, shifted, jnp.zeros_like(cs))
    return cs


def _splat16(vec, e):
    return _take16(vec, jnp.full((_L,), e, jnp.int32))


def _routing_body(sel_hbm, gate_hbm, cnt_hbm, rankl_hbm,
                  dispc_hbm, gsc_hbm, comb_hbm,
                  cnt_v, sel_v, gate_v, rankl_v, comb_v, tok2d, g2d, d2d,
                  neg_v, disp_sh, gsh):
    wid = lax.axis_index("s") * _NC + lax.axis_index("c")
    base_tok = wid * _CHUNK
    lane = lax.iota(jnp.int32, _L)
    zero = jnp.zeros((_L,), jnp.int32)

    for i in range((T // _NS) // _L):
        neg_v[pl.ds(i * _L, _L)] = jnp.full((_L,), -1, jnp.int32)
    pltpu.sync_copy(
        neg_v, disp_sh.at[pl.ds(lax.axis_index("s") * (T // _NS), T // _NS)])

    pltpu.sync_copy(cnt_hbm, cnt_v)
    pltpu.sync_copy(sel_hbm.at[pl.ds(base_tok, _CHUNK)], sel_v)
    pltpu.sync_copy(gate_hbm.at[pl.ds(base_tok, _CHUNK)], gate_v)
    pltpu.sync_copy(rankl_hbm.at[pl.ds(base_tok, _CHUNK)], rankl_v)

    base = zero
    tot = zero
    for w in range(_NW):
        row = cnt_v[w]
        base = base + jnp.where(w < wid, row, zero)
        tot = tot + row

    expert_lane = lane < NE
    drops = jnp.maximum(tot - CAP, 0)
    pads = jnp.where(expert_lane, jnp.maximum(CAP - tot, 0), 0)
    drop_excl = _cumsum16(drops) - drops
    pad_incl = _cumsum16(pads)
    pad_excl = pad_incl - pads

    for v in range(_NV):
        s = sel_v[pl.ds(v * _L, _L)]
        g = gate_v[pl.ds(v * _L, _L)]
        tok = base_tok + v * _L + lane
        rank = _take16(base, s) + rankl_v[pl.ds(v * _L, _L)]

        kept = rank < CAP
        kslot = s * CAP + rank
        kd = _take16(drop_excl, s) + (rank - CAP)
        ep = zero
        for e in range(NE):
            ep = ep + jnp.where(kd >= _splat16(pad_incl, e),
                                jnp.ones_like(kd), jnp.zeros_like(kd))
        ep = jnp.minimum(ep, NE - 1)
        slot_pad = ep * CAP + _take16(tot, ep) + (kd - _take16(pad_excl, ep))
        d = jnp.where(kept, kslot, slot_pad)

        j, c = v // (_NV // 2), (v % (_NV // 2)) * _L
        comb_v[pl.ds(v * _L, _L)] = d
        d2d[j, pl.ds(c, _L)] = d
        tok2d[j, pl.ds(c, _L)] = tok
        g2d[j, pl.ds(c, _L)] = jnp.where(kept, g, 0.0)

    plsc.subcore_barrier()
    for j in range(2):
        pltpu.sync_copy(tok2d.at[j], disp_sh.at[d2d.at[j]])
        pltpu.sync_copy(g2d.at[j], gsh.at[d2d.at[j]])
    plsc.subcore_barrier()
    sid = lax.axis_index("s")
    cid = lax.axis_index("c")
    stripe = sid * (T // _NS)
    out_off = cid * T + stripe
    pltpu.sync_copy(disp_sh.at[pl.ds(stripe, T // _NS)],
                    dispc_hbm.at[pl.ds(out_off, T // _NS)])
    pltpu.sync_copy(gsh.at[pl.ds(stripe, T // _NS)],
                    gsc_hbm.at[pl.ds(out_off, T // _NS)])
    pltpu.sync_copy(comb_v, comb_hbm.at[pl.ds(base_tok, _CHUNK)])


def _sc_route(sel, gate, counts, rankl):
    f = functools.partial(
        pl.kernel,
        out_type=[
            jax.ShapeDtypeStruct((2 * T,), jnp.int32),
            jax.ShapeDtypeStruct((2 * T,), jnp.float32),
            jax.ShapeDtypeStruct((T,), jnp.int32),
        ],
        mesh=plsc.VectorSubcoreMesh(**_MESH),
        scratch_types=[
            pltpu.VMEM((_NW, 16), jnp.int32),
            pltpu.VMEM((_CHUNK,), jnp.int32),
            pltpu.VMEM((_CHUNK,), jnp.float32),
            pltpu.VMEM((_CHUNK,), jnp.int32),
            pltpu.VMEM((_CHUNK,), jnp.int32),
            pltpu.VMEM((2, 128), jnp.int32),
            pltpu.VMEM((2, 128), jnp.float32),
            pltpu.VMEM((2, 128), jnp.int32),
            pltpu.VMEM((T // _NS,), jnp.int32),
            pltpu.VMEM_SHARED((T,), jnp.int32),
            pltpu.VMEM_SHARED((T,), jnp.float32),
        ],
    )(_routing_body)
    return f(sel, gate, counts, rankl)


_GC = 64


def _gather_body(src_hbm, idx_hbm, out_hbm, idx_v, rows_v, sem):
    wid = lax.axis_index("s") * _NC + lax.axis_index("c")
    base = wid * _CHUNK
    pltpu.sync_copy(idx_hbm.at[pl.ds(base, _CHUNK)], idx_v)
    for c in range(_CHUNK // _GC):
        pltpu.async_copy(
            src_hbm.at[idx_v.at[pl.ds(c * _GC, _GC)]], rows_v, sem).wait()
        pltpu.sync_copy(rows_v, out_hbm.at[pl.ds(base + c * _GC, _GC)])


def _sc_gather_rows(src, idx):
    f = functools.partial(
        pl.kernel,
        out_type=jax.ShapeDtypeStruct((T, EMBED), jnp.float32),
        mesh=plsc.VectorSubcoreMesh(**_MESH),
        scratch_types=[
            pltpu.VMEM((_CHUNK,), jnp.int32),
            pltpu.VMEM((_GC, EMBED), jnp.float32),
            pltpu.SemaphoreType.DMA,
        ],
    )(_gather_body)
    return f(src, idx)


def _mgather_body(src_hbm, dispc_hbm, gsc_hbm, out_hbm, gout_hbm,
                  d0_v, d1_v, g0_v, g1_v, idx_v, gm_v, rows_v, sem):
    wid = lax.axis_index("s") * _NC + lax.axis_index("c")
    base = wid * _CHUNK
    pltpu.sync_copy(dispc_hbm.at[pl.ds(base, _CHUNK)], d0_v)
    pltpu.sync_copy(dispc_hbm.at[pl.ds(T + base, _CHUNK)], d1_v)
    pltpu.sync_copy(gsc_hbm.at[pl.ds(base, _CHUNK)], g0_v)
    pltpu.sync_copy(gsc_hbm.at[pl.ds(T + base, _CHUNK)], g1_v)
    for v in range(_NV):
        sl = pl.ds(v * _L, _L)
        d0 = d0_v[sl]
        ok0 = d0 >= 0
        idx_v[sl] = jnp.where(ok0, d0, d1_v[sl])
        gm_v[sl] = jnp.where(ok0, g0_v[sl], g1_v[sl])
    pltpu.sync_copy(gm_v, gout_hbm.at[pl.ds(base, _CHUNK)])
    for c in range(_CHUNK // _GC):
        pltpu.async_copy(
            src_hbm.at[idx_v.at[pl.ds(c * _GC, _GC)]], rows_v, sem).wait()
        pltpu.sync_copy(rows_v, out_hbm.at[pl.ds(base + c * _GC, _GC)])


def _sc_merge_gather(src, dispc, gsc):
    f = functools.partial(
        pl.kernel,
        out_type=[
            jax.ShapeDtypeStruct((T, EMBED), jnp.float32),
            jax.ShapeDtypeStruct((T,), jnp.float32),
        ],
        mesh=plsc.VectorSubcoreMesh(**_MESH),
        scratch_types=[
            pltpu.VMEM((_CHUNK,), jnp.int32),
            pltpu.VMEM((_CHUNK,), jnp.int32),
            pltpu.VMEM((_CHUNK,), jnp.float32),
            pltpu.VMEM((_CHUNK,), jnp.float32),
            pltpu.VMEM((_CHUNK,), jnp.int32),
            pltpu.VMEM((_CHUNK,), jnp.float32),
            pltpu.VMEM((_GC, EMBED), jnp.float32),
            pltpu.SemaphoreType.DMA,
        ],
    )(_mgather_body)
    return f(src, dispc, gsc)


def kernel(x, Wr1, br1, Wr2, br2, W1, b1, W2, b2):
    sel2, gate2, counts16, rankl2 = _router(x, Wr1, br1, Wr2, br2)
    dispc, gsc, comb = _sc_route(sel2[:, 0], gate2[:, 0],
                                 counts16.reshape(_NW, 16), rankl2[:, 0])
    x_sorted, gsort = _sc_merge_gather(x, dispc, gsc)
    y = _ffn(x_sorted, W1, b1, W2, b2, gsort[:, None])
    return _sc_gather_rows(y, comb)


# speedup vs baseline: 1.1556x; 1.0292x over previous
import functools

import jax
import jax.numpy as jnp
from jax import lax
from jax.experimental import pallas as pl
from jax.experimental.pallas import tpu as pltpu
from jax.experimental.pallas import tpu_sc as plsc

EMBED = 1024
NE = 8
CAP = 1024
DFF = 4096
T = 8192

_HI = jax.lax.Precision.DEFAULT


_RBM = 256


def _router_body(x_ref, wr1_ref, br1_ref, wr2_ref, br2_ref,
                 sel_ref, gate_ref, cnt_ref, rankl_ref):
    h = jnp.maximum(
        jnp.dot(x_ref[...], wr1_ref[...], precision=_HI) + br1_ref[...], 0.0)
    logits = jnp.dot(h, wr2_ref[...], precision=_HI) + br2_ref[...]
    lmax = jnp.max(logits, axis=1, keepdims=True)
    iota8 = jax.lax.broadcasted_iota(jnp.int32, logits.shape, 1)
    sel = jnp.min(jnp.where(logits >= lmax, iota8, NE), axis=1)
    gate = 1.0 / jnp.sum(jnp.exp(logits - lmax), axis=1)
    sel_ref[...] = sel[:, None]
    gate_ref[...] = gate[:, None]
    onehot = (sel[:, None] == jax.lax.broadcasted_iota(
        jnp.int32, (_RBM, 16), 1)).astype(jnp.int32)
    cnt_ref[...] = jnp.sum(onehot, axis=0, keepdims=True)[None]
    cum = onehot
    sh = 1
    while sh < _RBM:
        cum = cum + jnp.concatenate(
            [jnp.zeros((sh, 16), jnp.int32), cum[:-sh]], axis=0)
        sh *= 2
    rankl_ref[...] = jnp.sum(onehot * (cum - 1), axis=1)[:, None]


def _router(x, Wr1, br1, Wr2, br2):
    grid = (T // _RBM,)
    return pl.pallas_call(
        _router_body,
        grid=grid,
        in_specs=[
            pl.BlockSpec((_RBM, EMBED), lambda i: (i, 0)),
            pl.BlockSpec((EMBED, EMBED), lambda i: (0, 0)),
            pl.BlockSpec((1, EMBED), lambda i: (0, 0)),
            pl.BlockSpec((EMBED, NE), lambda i: (0, 0)),
            pl.BlockSpec((1, NE), lambda i: (0, 0)),
        ],
        out_specs=[
            pl.BlockSpec((_RBM, 1), lambda i: (i, 0)),
            pl.BlockSpec((_RBM, 1), lambda i: (i, 0)),
            pl.BlockSpec((1, 1, 16), lambda i: (i, 0, 0)),
            pl.BlockSpec((_RBM, 1), lambda i: (i, 0)),
        ],
        out_shape=[
            jax.ShapeDtypeStruct((T, 1), jnp.int32),
            jax.ShapeDtypeStruct((T, 1), jnp.float32),
            jax.ShapeDtypeStruct((T // _RBM, 1, 16), jnp.int32),
            jax.ShapeDtypeStruct((T, 1), jnp.int32),
        ],
    )(x, Wr1, br1.reshape(1, EMBED), Wr2, br2.reshape(1, NE))


_BF = 2048
_NF = DFF // _BF


def _ffn_body(xs_ref, w1_ref, b1_ref, w2_ref, b2_ref, g_ref, out_ref):
    f = pl.program_id(1)
    h = jnp.maximum(
        jnp.dot(xs_ref[...], w1_ref[0], precision=_HI) + b1_ref[0], 0.0)
    contrib = jnp.dot(h, w2_ref[0], precision=_HI)

    @pl.when(f == 0)
    def _():
        out_ref[...] = contrib + b2_ref[0]

    @pl.when(f != 0)
    def _():
        out_ref[...] = out_ref[...] + contrib

    @pl.when(f == _NF - 1)
    def _():
        out_ref[...] = out_ref[...] * g_ref[...]


def _ffn(x_sorted, W1, b1, W2, b2, gate_sorted):
    grid = (NE, _NF)
    return pl.pallas_call(
        _ffn_body,
        grid=grid,
        in_specs=[
            pl.BlockSpec((CAP, EMBED), lambda e, f: (e, 0)),
            pl.BlockSpec((1, EMBED, _BF), lambda e, f: (e, 0, f)),
            pl.BlockSpec((1, 1, _BF), lambda e, f: (e, 0, f)),
            pl.BlockSpec((1, _BF, EMBED), lambda e, f: (e, f, 0)),
            pl.BlockSpec((1, 1, EMBED), lambda e, f: (e, 0, 0)),
            pl.BlockSpec((CAP, 1), lambda e, f: (e, 0)),
        ],
        out_specs=pl.BlockSpec((CAP, EMBED), lambda e, f: (e, 0)),
        out_shape=jax.ShapeDtypeStruct((T, EMBED), jnp.float32),
        compiler_params=pltpu.CompilerParams(
            dimension_semantics=("parallel", "arbitrary")),
    )(x_sorted, W1, b1.reshape(NE, 1, DFF), W2, b2.reshape(NE, 1, EMBED),
      gate_sorted)


_NC = 2
_NS = 16
_NW = _NC * _NS
_L = 16
_CHUNK = T // _NW
_NV = _CHUNK // _L

_MESH = dict(core_axis_name="c", subcore_axis_name="s")


def _take16(vec, idx):
    return vec.at[idx].get(mode="promise_in_bounds")


def _cumsum16(x):
    lane = lax.iota(jnp.int32, _L)
    cs = x
    for sh in (1, 2, 4, 8):
        shifted = _take16(cs, jnp.maximum(lane - sh, 0))
        cs = cs + jnp.where(lane >= sh, shifted, jnp.zeros_like(cs))
    return cs


def _splat16(vec, e):
    return _take16(vec, jnp.full((_L,), e, jnp.int32))


def _routing_body(sel_hbm, gate_hbm, cnt_hbm, rankl_hbm,
                  dispc_hbm, gsc_hbm, comb_hbm,
                  cnt_v, sel_v, gate_v, rankl_v, comb_v, tok2d, g2d, d2d,
                  neg_v, disp_sh, gsh):
    wid = lax.axis_index("s") * _NC + lax.axis_index("c")
    base_tok = wid * _CHUNK
    lane = lax.iota(jnp.int32, _L)
    zero = jnp.zeros((_L,), jnp.int32)

    for i in range((T // _NS) // _L):
        neg_v[pl.ds(i * _L, _L)] = jnp.full((_L,), -1, jnp.int32)
    pltpu.sync_copy(
        neg_v, disp_sh.at[pl.ds(lax.axis_index("s") * (T // _NS), T // _NS)])

    pltpu.sync_copy(cnt_hbm, cnt_v)
    pltpu.sync_copy(sel_hbm.at[pl.ds(base_tok, _CHUNK)], sel_v)
    pltpu.sync_copy(gate_hbm.at[pl.ds(base_tok, _CHUNK)], gate_v)
    pltpu.sync_copy(rankl_hbm.at[pl.ds(base_tok, _CHUNK)], rankl_v)

    base = zero
    tot = zero
    for w in range(_NW):
        row = cnt_v[w]
        base = base + jnp.where(w < wid, row, zero)
        tot = tot + row

    expert_lane = lane < NE
    drops = jnp.maximum(tot - CAP, 0)
    pads = jnp.where(expert_lane, jnp.maximum(CAP - tot, 0), 0)
    drop_excl = _cumsum16(drops) - drops
    pad_incl = _cumsum16(pads)
    pad_excl = pad_incl - pads

    for v in range(_NV):
        s = sel_v[pl.ds(v * _L, _L)]
        g = gate_v[pl.ds(v * _L, _L)]
        tok = base_tok + v * _L + lane
        rank = _take16(base, s) + rankl_v[pl.ds(v * _L, _L)]

        kept = rank < CAP
        kslot = s * CAP + rank
        kd = _take16(drop_excl, s) + (rank - CAP)
        ep = zero
        for e in range(NE):
            ep = ep + jnp.where(kd >= _splat16(pad_incl, e),
                                jnp.ones_like(kd), jnp.zeros_like(kd))
        ep = jnp.minimum(ep, NE - 1)
        slot_pad = ep * CAP + _take16(tot, ep) + (kd - _take16(pad_excl, ep))
        d = jnp.where(kept, kslot, slot_pad)

        j, c = v // (_NV // 2), (v % (_NV // 2)) * _L
        comb_v[pl.ds(v * _L, _L)] = d
        d2d[j, pl.ds(c, _L)] = d
        tok2d[j, pl.ds(c, _L)] = tok
        g2d[j, pl.ds(c, _L)] = jnp.where(kept, g, 0.0)

    plsc.subcore_barrier()
    for j in range(2):
        pltpu.sync_copy(tok2d.at[j], disp_sh.at[d2d.at[j]])
        pltpu.sync_copy(g2d.at[j], gsh.at[d2d.at[j]])
    plsc.subcore_barrier()
    sid = lax.axis_index("s")
    cid = lax.axis_index("c")
    stripe = sid * (T // _NS)
    out_off = cid * T + stripe
    pltpu.sync_copy(disp_sh.at[pl.ds(stripe, T // _NS)],
                    dispc_hbm.at[pl.ds(out_off, T // _NS)])
    pltpu.sync_copy(gsh.at[pl.ds(stripe, T // _NS)],
                    gsc_hbm.at[pl.ds(out_off, T // _NS)])
    pltpu.sync_copy(comb_v, comb_hbm.at[pl.ds(base_tok, _CHUNK)])


def _sc_route(sel, gate, counts, rankl):
    f = functools.partial(
        pl.kernel,
        out_type=[
            jax.ShapeDtypeStruct((2 * T,), jnp.int32),
            jax.ShapeDtypeStruct((2 * T,), jnp.float32),
            jax.ShapeDtypeStruct((T,), jnp.int32),
        ],
        mesh=plsc.VectorSubcoreMesh(**_MESH),
        scratch_types=[
            pltpu.VMEM((_NW, 16), jnp.int32),
            pltpu.VMEM((_CHUNK,), jnp.int32),
            pltpu.VMEM((_CHUNK,), jnp.float32),
            pltpu.VMEM((_CHUNK,), jnp.int32),
            pltpu.VMEM((_CHUNK,), jnp.int32),
            pltpu.VMEM((2, 128), jnp.int32),
            pltpu.VMEM((2, 128), jnp.float32),
            pltpu.VMEM((2, 128), jnp.int32),
            pltpu.VMEM((T // _NS,), jnp.int32),
            pltpu.VMEM_SHARED((T,), jnp.int32),
            pltpu.VMEM_SHARED((T,), jnp.float32),
        ],
    )(_routing_body)
    return f(sel, gate, counts, rankl)


_GC = 64


def _gather_body(src_hbm, idx_hbm, out_hbm, idx_v, rows_v, sem):
    wid = lax.axis_index("s") * _NC + lax.axis_index("c")
    base = wid * _CHUNK
    pltpu.sync_copy(idx_hbm.at[pl.ds(base, _CHUNK)], idx_v)
    for c in range(_CHUNK // _GC):
        pltpu.async_copy(
            src_hbm.at[idx_v.at[pl.ds(c * _GC, _GC)]], rows_v, sem).wait()
        pltpu.sync_copy(rows_v, out_hbm.at[pl.ds(base + c * _GC, _GC)])


def _sc_gather_rows(src, idx):
    f = functools.partial(
        pl.kernel,
        out_type=jax.ShapeDtypeStruct((T, EMBED), jnp.float32),
        mesh=plsc.VectorSubcoreMesh(**_MESH),
        scratch_types=[
            pltpu.VMEM((_CHUNK,), jnp.int32),
            pltpu.VMEM((_GC, EMBED), jnp.float32),
            pltpu.SemaphoreType.DMA,
        ],
    )(_gather_body)
    return f(src, idx)


def _mgather_body(src_hbm, dispc_hbm, gsc_hbm, out_hbm, gout_hbm,
                  d0_v, d1_v, g0_v, g1_v, idx_v, gm_v, rows_v, sem):
    wid = lax.axis_index("s") * _NC + lax.axis_index("c")
    base = wid * _CHUNK
    pltpu.sync_copy(dispc_hbm.at[pl.ds(base, _CHUNK)], d0_v)
    pltpu.sync_copy(dispc_hbm.at[pl.ds(T + base, _CHUNK)], d1_v)
    pltpu.sync_copy(gsc_hbm.at[pl.ds(base, _CHUNK)], g0_v)
    pltpu.sync_copy(gsc_hbm.at[pl.ds(T + base, _CHUNK)], g1_v)
    for v in range(_NV):
        sl = pl.ds(v * _L, _L)
        d0 = d0_v[sl]
        ok0 = d0 >= 0
        idx_v[sl] = jnp.where(ok0, d0, d1_v[sl])
        gm_v[sl] = jnp.where(ok0, g0_v[sl], g1_v[sl])
    pltpu.sync_copy(gm_v, gout_hbm.at[pl.ds(base, _CHUNK)])
    for c in range(_CHUNK // _GC):
        pltpu.async_copy(
            src_hbm.at[idx_v.at[pl.ds(c * _GC, _GC)]], rows_v, sem).wait()
        pltpu.sync_copy(rows_v, out_hbm.at[pl.ds(base + c * _GC, _GC)])


def _sc_merge_gather(src, dispc, gsc):
    f = functools.partial(
        pl.kernel,
        out_type=[
            jax.ShapeDtypeStruct((T, EMBED), jnp.float32),
            jax.ShapeDtypeStruct((T,), jnp.float32),
        ],
        mesh=plsc.VectorSubcoreMesh(**_MESH),
        scratch_types=[
            pltpu.VMEM((_CHUNK,), jnp.int32),
            pltpu.VMEM((_CHUNK,), jnp.int32),
            pltpu.VMEM((_CHUNK,), jnp.float32),
            pltpu.VMEM((_CHUNK,), jnp.float32),
            pltpu.VMEM((_CHUNK,), jnp.int32),
            pltpu.VMEM((_CHUNK,), jnp.float32),
            pltpu.VMEM((_GC, EMBED), jnp.float32),
            pltpu.SemaphoreType.DMA,
        ],
    )(_mgather_body)
    return f(src, dispc, gsc)


def kernel(x, Wr1, br1, Wr2, br2, W1, b1, W2, b2):
    sel2, gate2, counts16, rankl2 = _router(x, Wr1, br1, Wr2, br2)
    dispc, gsc, comb = _sc_route(sel2[:, 0], gate2[:, 0],
                                 counts16.reshape(_NW, 16), rankl2[:, 0])
    x_sorted, gsort = _sc_merge_gather(x, dispc, gsc)
    y = _ffn(x_sorted, W1, b1, W2, b2, gsort[:, None])
    return _sc_gather_rows(y, comb)


# speedup vs baseline: 1.1638x; 1.0071x over previous
import functools

import jax
import jax.numpy as jnp
from jax import lax
from jax.experimental import pallas as pl
from jax.experimental.pallas import tpu as pltpu
from jax.experimental.pallas import tpu_sc as plsc

EMBED = 1024
NE = 8
CAP = 1024
DFF = 4096
T = 8192

_HI = jax.lax.Precision.DEFAULT


_RBM = 256


def _router_body(x_ref, wr1_ref, br1_ref, wr2_ref, br2_ref,
                 sel_ref, gate_ref, cnt_ref, rankl_ref):
    h = jnp.maximum(
        jnp.dot(x_ref[...], wr1_ref[...], precision=_HI) + br1_ref[...], 0.0)
    logits = jnp.dot(h, wr2_ref[...], precision=_HI) + br2_ref[...]
    lmax = jnp.max(logits, axis=1, keepdims=True)
    iota8 = jax.lax.broadcasted_iota(jnp.int32, logits.shape, 1)
    sel = jnp.min(jnp.where(logits >= lmax, iota8, NE), axis=1)
    gate = 1.0 / jnp.sum(jnp.exp(logits - lmax), axis=1)
    sel_ref[...] = sel[:, None]
    gate_ref[...] = gate[:, None]
    onehot = (sel[:, None] == jax.lax.broadcasted_iota(
        jnp.int32, (_RBM, 16), 1)).astype(jnp.int32)
    cnt_ref[...] = jnp.sum(onehot, axis=0, keepdims=True)[None]
    cum = onehot
    sh = 1
    while sh < _RBM:
        cum = cum + jnp.concatenate(
            [jnp.zeros((sh, 16), jnp.int32), cum[:-sh]], axis=0)
        sh *= 2
    rankl_ref[...] = jnp.sum(onehot * (cum - 1), axis=1)[:, None]


def _router(x, Wr1, br1, Wr2, br2):
    grid = (T // _RBM,)
    return pl.pallas_call(
        _router_body,
        grid=grid,
        in_specs=[
            pl.BlockSpec((_RBM, EMBED), lambda i: (i, 0)),
            pl.BlockSpec((EMBED, EMBED), lambda i: (0, 0)),
            pl.BlockSpec((1, EMBED), lambda i: (0, 0)),
            pl.BlockSpec((EMBED, NE), lambda i: (0, 0)),
            pl.BlockSpec((1, NE), lambda i: (0, 0)),
        ],
        out_specs=[
            pl.BlockSpec((_RBM, 1), lambda i: (i, 0)),
            pl.BlockSpec((_RBM, 1), lambda i: (i, 0)),
            pl.BlockSpec((1, 1, 16), lambda i: (i, 0, 0)),
            pl.BlockSpec((_RBM, 1), lambda i: (i, 0)),
        ],
        out_shape=[
            jax.ShapeDtypeStruct((T, 1), jnp.int32),
            jax.ShapeDtypeStruct((T, 1), jnp.float32),
            jax.ShapeDtypeStruct((T // _RBM, 1, 16), jnp.int32),
            jax.ShapeDtypeStruct((T, 1), jnp.int32),
        ],
    )(x, Wr1, br1.reshape(1, EMBED), Wr2, br2.reshape(1, NE))


_BF = 2048
_NF = DFF // _BF


def _ffn_body(xs_ref, w1_ref, b1_ref, w2_ref, b2_ref, g_ref, out_ref):
    f = pl.program_id(1)
    h = jnp.maximum(
        jnp.dot(xs_ref[...], w1_ref[0], precision=_HI) + b1_ref[0], 0.0)
    contrib = jnp.dot(h, w2_ref[0], precision=_HI)

    @pl.when(f == 0)
    def _():
        out_ref[...] = contrib + b2_ref[0]

    @pl.when(f != 0)
    def _():
        out_ref[...] = out_ref[...] + contrib

    @pl.when(f == _NF - 1)
    def _():
        out_ref[...] = out_ref[...] * g_ref[...]


def _ffn(x_sorted, W1, b1, W2, b2, gate_sorted):
    grid = (NE, _NF)
    return pl.pallas_call(
        _ffn_body,
        grid=grid,
        in_specs=[
            pl.BlockSpec((CAP, EMBED), lambda e, f: (e, 0)),
            pl.BlockSpec((1, EMBED, _BF), lambda e, f: (e, 0, f)),
            pl.BlockSpec((1, 1, _BF), lambda e, f: (e, 0, f)),
            pl.BlockSpec((1, _BF, EMBED), lambda e, f: (e, f, 0)),
            pl.BlockSpec((1, 1, EMBED), lambda e, f: (e, 0, 0)),
            pl.BlockSpec((CAP, 1), lambda e, f: (e, 0)),
        ],
        out_specs=pl.BlockSpec((CAP, EMBED), lambda e, f: (e, 0)),
        out_shape=jax.ShapeDtypeStruct((T, EMBED), jnp.float32),
        compiler_params=pltpu.CompilerParams(
            dimension_semantics=("parallel", "arbitrary")),
    )(x_sorted, W1, b1.reshape(NE, 1, DFF), W2, b2.reshape(NE, 1, EMBED),
      gate_sorted)


_NC = 2
_NS = 16
_NW = _NC * _NS
_L = 16
_CHUNK = T // _NW
_NV = _CHUNK // _L

_MESH = dict(core_axis_name="c", subcore_axis_name="s")


def _take16(vec, idx):
    return vec.at[idx].get(mode="promise_in_bounds")


def _cumsum16(x):
    lane = lax.iota(jnp.int32, _L)
    cs = x
    for sh in (1, 2, 4, 8):
        shifted = _take16(cs, jnp.maximum(lane - sh, 0))
        cs = cs + jnp.where(lane >= sh, shifted, jnp.zeros_like(cs))
    return cs


def _splat16(vec, e):
    return _take16(vec, jnp.full((_L,), e, jnp.int32))


def _routing_body(sel_hbm, gate_hbm, cnt_hbm, rankl_hbm,
                  dispc_hbm, gsc_hbm, comb_hbm,
                  cnt_v, sel_v, gate_v, rankl_v, comb_v, tok2d, g2d, d2d,
                  neg_v, disp_sh, gsh):
    wid = lax.axis_index("s") * _NC + lax.axis_index("c")
    base_tok = wid * _CHUNK
    lane = lax.iota(jnp.int32, _L)
    zero = jnp.zeros((_L,), jnp.int32)

    for i in range((T // _NS) // _L):
        neg_v[pl.ds(i * _L, _L)] = jnp.full((_L,), -1, jnp.int32)
    pltpu.sync_copy(
        neg_v, disp_sh.at[pl.ds(lax.axis_index("s") * (T // _NS), T // _NS)])

    pltpu.sync_copy(cnt_hbm, cnt_v)
    pltpu.sync_copy(sel_hbm.at[pl.ds(base_tok, _CHUNK)], sel_v)
    pltpu.sync_copy(gate_hbm.at[pl.ds(base_tok, _CHUNK)], gate_v)
    pltpu.sync_copy(rankl_hbm.at[pl.ds(base_tok, _CHUNK)], rankl_v)

    base = zero
    tot = zero
    for w in range(_NW):
        row = cnt_v[w]
        base = base + jnp.where(w < wid, row, zero)
        tot = tot + row

    expert_lane = lane < NE
    drops = jnp.maximum(tot - CAP, 0)
    pads = jnp.where(expert_lane, jnp.maximum(CAP - tot, 0), 0)
    drop_excl = _cumsum16(drops) - drops
    pad_incl = _cumsum16(pads)
    pad_excl = pad_incl - pads

    for v in range(_NV):
        s = sel_v[pl.ds(v * _L, _L)]
        g = gate_v[pl.ds(v * _L, _L)]
        tok = base_tok + v * _L + lane
        rank = _take16(base, s) + rankl_v[pl.ds(v * _L, _L)]

        kept = rank < CAP
        kslot = s * CAP + rank
        kd = _take16(drop_excl, s) + (rank - CAP)
        ep = zero
        for e in range(NE):
            ep = ep + jnp.where(kd >= _splat16(pad_incl, e),
                                jnp.ones_like(kd), jnp.zeros_like(kd))
        ep = jnp.minimum(ep, NE - 1)
        slot_pad = ep * CAP + _take16(tot, ep) + (kd - _take16(pad_excl, ep))
        d = jnp.where(kept, kslot, slot_pad)

        j, c = v // (_NV // 2), (v % (_NV // 2)) * _L
        comb_v[pl.ds(v * _L, _L)] = d
        d2d[j, pl.ds(c, _L)] = d
        tok2d[j, pl.ds(c, _L)] = tok
        g2d[j, pl.ds(c, _L)] = jnp.where(kept, g, 0.0)

    plsc.subcore_barrier()
    for j in range(2):
        pltpu.sync_copy(tok2d.at[j], disp_sh.at[d2d.at[j]])
        pltpu.sync_copy(g2d.at[j], gsh.at[d2d.at[j]])
    plsc.subcore_barrier()
    sid = lax.axis_index("s")
    cid = lax.axis_index("c")
    stripe = sid * (T // _NS)
    out_off = cid * T + stripe
    pltpu.sync_copy(disp_sh.at[pl.ds(stripe, T // _NS)],
                    dispc_hbm.at[pl.ds(out_off, T // _NS)])
    pltpu.sync_copy(gsh.at[pl.ds(stripe, T // _NS)],
                    gsc_hbm.at[pl.ds(out_off, T // _NS)])
    pltpu.sync_copy(comb_v, comb_hbm.at[pl.ds(base_tok, _CHUNK)])


def _sc_route(sel, gate, counts, rankl):
    f = functools.partial(
        pl.kernel,
        out_type=[
            jax.ShapeDtypeStruct((2 * T,), jnp.int32),
            jax.ShapeDtypeStruct((2 * T,), jnp.float32),
            jax.ShapeDtypeStruct((T,), jnp.int32),
        ],
        mesh=plsc.VectorSubcoreMesh(**_MESH),
        scratch_types=[
            pltpu.VMEM((_NW, 16), jnp.int32),
            pltpu.VMEM((_CHUNK,), jnp.int32),
            pltpu.VMEM((_CHUNK,), jnp.float32),
            pltpu.VMEM((_CHUNK,), jnp.int32),
            pltpu.VMEM((_CHUNK,), jnp.int32),
            pltpu.VMEM((2, 128), jnp.int32),
            pltpu.VMEM((2, 128), jnp.float32),
            pltpu.VMEM((2, 128), jnp.int32),
            pltpu.VMEM((T // _NS,), jnp.int32),
            pltpu.VMEM_SHARED((T,), jnp.int32),
            pltpu.VMEM_SHARED((T,), jnp.float32),
        ],
    )(_routing_body)
    return f(sel, gate, counts, rankl)


_GC = 32
_NCH = _CHUNK // _GC


def _pipelined_gather(src_hbm, idx_v, out_hbm, base, buf_a, buf_b, sems):
    bufs = (buf_a, buf_b)
    g_descs = [None] * _NCH
    s_descs = [None] * _NCH

    def mk_gather(c):
        return pltpu.make_async_copy(
            src_hbm.at[idx_v.at[pl.ds(c * _GC, _GC)]], bufs[c % 2],
            sems.at[c % 2])

    g_descs[0] = mk_gather(0)
    g_descs[0].start()
    for c in range(_NCH):
        if c + 1 < _NCH:
            if c >= 1:
                s_descs[c - 1].wait()
            g_descs[c + 1] = mk_gather(c + 1)
            g_descs[c + 1].start()
        g_descs[c].wait()
        s_descs[c] = pltpu.make_async_copy(
            bufs[c % 2], out_hbm.at[pl.ds(base + c * _GC, _GC)],
            sems.at[2 + c % 2])
        s_descs[c].start()
    s_descs[_NCH - 2].wait()
    s_descs[_NCH - 1].wait()


def _gather_body(src_hbm, idx_hbm, out_hbm, idx_v, rows_a, rows_b, sems):
    wid = lax.axis_index("s") * _NC + lax.axis_index("c")
    base = wid * _CHUNK
    pltpu.sync_copy(idx_hbm.at[pl.ds(base, _CHUNK)], idx_v)
    _pipelined_gather(src_hbm, idx_v, out_hbm, base, rows_a, rows_b, sems)


def _sc_gather_rows(src, idx):
    f = functools.partial(
        pl.kernel,
        out_type=jax.ShapeDtypeStruct((T, EMBED), jnp.float32),
        mesh=plsc.VectorSubcoreMesh(**_MESH),
        scratch_types=[
            pltpu.VMEM((_CHUNK,), jnp.int32),
            pltpu.VMEM((_GC, EMBED), jnp.float32),
            pltpu.VMEM((_GC, EMBED), jnp.float32),
            pltpu.SemaphoreType.DMA((4,)),
        ],
    )(_gather_body)
    return f(src, idx)


def _mgather_body(src_hbm, dispc_hbm, gsc_hbm, out_hbm, gout_hbm,
                  d0_v, d1_v, g0_v, g1_v, idx_v, gm_v, rows_a, rows_b, sems):
    wid = lax.axis_index("s") * _NC + lax.axis_index("c")
    base = wid * _CHUNK
    pltpu.sync_copy(dispc_hbm.at[pl.ds(base, _CHUNK)], d0_v)
    pltpu.sync_copy(dispc_hbm.at[pl.ds(T + base, _CHUNK)], d1_v)
    pltpu.sync_copy(gsc_hbm.at[pl.ds(base, _CHUNK)], g0_v)
    pltpu.sync_copy(gsc_hbm.at[pl.ds(T + base, _CHUNK)], g1_v)
    for v in range(_NV):
        sl = pl.ds(v * _L, _L)
        d0 = d0_v[sl]
        ok0 = d0 >= 0
        idx_v[sl] = jnp.where(ok0, d0, d1_v[sl])
        gm_v[sl] = jnp.where(ok0, g0_v[sl], g1_v[sl])
    pltpu.sync_copy(gm_v, gout_hbm.at[pl.ds(base, _CHUNK)])
    _pipelined_gather(src_hbm, idx_v, out_hbm, base, rows_a, rows_b, sems)


def _sc_merge_gather(src, dispc, gsc):
    f = functools.partial(
        pl.kernel,
        out_type=[
            jax.ShapeDtypeStruct((T, EMBED), jnp.float32),
            jax.ShapeDtypeStruct((T,), jnp.float32),
        ],
        mesh=plsc.VectorSubcoreMesh(**_MESH),
        scratch_types=[
            pltpu.VMEM((_CHUNK,), jnp.int32),
            pltpu.VMEM((_CHUNK,), jnp.int32),
            pltpu.VMEM((_CHUNK,), jnp.float32),
            pltpu.VMEM((_CHUNK,), jnp.float32),
            pltpu.VMEM((_CHUNK,), jnp.int32),
            pltpu.VMEM((_CHUNK,), jnp.float32),
            pltpu.VMEM((_GC, EMBED), jnp.float32),
            pltpu.VMEM((_GC, EMBED), jnp.float32),
            pltpu.SemaphoreType.DMA((4,)),
        ],
    )(_mgather_body)
    return f(src, dispc, gsc)


def kernel(x, Wr1, br1, Wr2, br2, W1, b1, W2, b2):
    sel2, gate2, counts16, rankl2 = _router(x, Wr1, br1, Wr2, br2)
    dispc, gsc, comb = _sc_route(sel2[:, 0], gate2[:, 0],
                                 counts16.reshape(_NW, 16), rankl2[:, 0])
    x_sorted, gsort = _sc_merge_gather(x, dispc, gsc)
    y = _ffn(x_sorted, W1, b1, W2, b2, gsort[:, None])
    return _sc_gather_rows(y, comb)
